# baseline (speedup 1.0000x reference)
import functools

import jax
import jax.numpy as jnp
from jax import lax
from jax.experimental import pallas as pl
from jax.experimental.pallas import tpu as pltpu
from jax.experimental.pallas import tpu_sc as plsc

VOCAB = 100000
BASIC_DIM = 300
MODIF_DIM = 100
FUSED_DIM = 400
PAD_DIM = 512
BATCH = 16384
HIST = 50
HIST_P = 56

_info = plsc.get_sparse_core_info()
NC = _info.num_cores
NS = _info.num_subcores
NW = NC * NS
SLABS_W = BATCH // NW
NBUF = 4
KB = 64
NBLK = SLABS_W // KB

_mesh = plsc.VectorSubcoreMesh(core_axis_name="c", subcore_axis_name="s")


@functools.partial(
    pl.kernel,
    mesh=_mesh,
    compiler_params=pltpu.CompilerParams(use_tc_tiling_on_sc=True),
    out_type=jax.ShapeDtypeStruct((BATCH, HIST_P, PAD_DIM), jnp.float32),
    scratch_types=[
        pltpu.VMEM((KB, HIST_P), jnp.int32),
        pltpu.VMEM((NBUF, HIST_P, PAD_DIM), jnp.float32),
        pltpu.SemaphoreType.DMA,
        pltpu.SemaphoreType.DMA,
        pltpu.SemaphoreType.DMA,
        pltpu.SemaphoreType.DMA,
        pltpu.SemaphoreType.DMA,
    ],
)
def _gather(table_hbm, ids_hbm, out_hbm, idx_v, rows_v,
            sem_g, sw0, sw1, sw2, sw3):
    w = lax.axis_index("s") * NC + lax.axis_index("c")
    base = w * SLABS_W
    sems_w = (sw0, sw1, sw2, sw3)

    def slab_step(s, bi):
        b = base + s
        @pl.when(s >= NBUF)
        def _():
            pltpu.make_async_copy(
                rows_v.at[bi], out_hbm.at[b - NBUF], sems_w[bi]).wait()

        pltpu.async_copy(
            table_hbm.at[idx_v.at[s % KB]], rows_v.at[bi], sem_g).wait()
        pltpu.async_copy(rows_v.at[bi], out_hbm.at[b], sems_w[bi])

    def inner(i, blk):
        for r in range(NBUF):
            slab_step(blk * KB + i * NBUF + r, r)
        return blk

    def block(blk, carry):
        pltpu.sync_copy(ids_hbm.at[pl.ds(base + blk * KB, KB)], idx_v)
        lax.fori_loop(0, KB // NBUF, inner, blk, unroll=False)
        return carry

    lax.fori_loop(0, NBLK, block, 0, unroll=False)

    for r in range(NBUF):
        b = base + SLABS_W - NBUF + r
        pltpu.make_async_copy(
            rows_v.at[r], out_hbm.at[b], sems_w[r]).wait()


def kernel(word_ids, basic, modif):
    fused = jnp.concatenate(
        [basic, modif,
         jnp.zeros((VOCAB, PAD_DIM - FUSED_DIM), jnp.float32)], axis=1)
    ids_p = jnp.pad(word_ids, ((0, 0), (0, HIST_P - HIST)))
    return _gather(fused, ids_p)[:, :HIST, :FUSED_DIM]


# speedup vs baseline: 1.9872x; 1.9872x over previous
import functools

import jax
import jax.numpy as jnp
from jax import lax
from jax.experimental import pallas as pl
from jax.experimental.pallas import tpu as pltpu
from jax.experimental.pallas import tpu_sc as plsc

VOCAB = 100000
BASIC_DIM = 300
MODIF_DIM = 100
FUSED_DIM = BASIC_DIM + MODIF_DIM
BATCH = 16384
HIST = 50
N = BATCH * HIST

_info = plsc.get_sparse_core_info()
NC = _info.num_cores
NS = _info.num_subcores
NW = NC * NS
PER_W = N // NW
CHUNK = 128
NCHUNK = PER_W // CHUNK
NBUF = 2

_mesh = plsc.VectorSubcoreMesh(core_axis_name="c", subcore_axis_name="s")


@functools.partial(
    pl.kernel,
    mesh=_mesh,
    compiler_params=pltpu.CompilerParams(use_tc_tiling_on_sc=False),
    out_type=jax.ShapeDtypeStruct((N, FUSED_DIM), jnp.float32),
    scratch_types=[
        pltpu.VMEM((PER_W,), jnp.int32),
        pltpu.VMEM((NBUF, CHUNK, FUSED_DIM), jnp.float32),
        pltpu.SemaphoreType.DMA,
        pltpu.SemaphoreType.DMA,
        pltpu.SemaphoreType.DMA,
    ],
)
def _gather(table_hbm, idx_hbm, out_hbm, idx_v, rows_v, sem_g, sem_w0, sem_w1):
    wid = lax.axis_index("s") * NC + lax.axis_index("c")
    base = wid * PER_W
    sems_w = (sem_w0, sem_w1)

    pltpu.sync_copy(idx_hbm.at[pl.ds(base, PER_W)], idx_v)

    def chunk_step(c, b):
        @pl.when(c >= NBUF)
        def _():
            pltpu.make_async_copy(
                rows_v.at[b],
                out_hbm.at[pl.ds(base + (c - NBUF) * CHUNK, CHUNK)],
                sems_w[b],
            ).wait()

        pltpu.async_copy(
            table_hbm.at[idx_v.at[pl.ds(c * CHUNK, CHUNK)]],
            rows_v.at[b],
            sem_g,
        ).wait()
        pltpu.async_copy(
            rows_v.at[b],
            out_hbm.at[pl.ds(base + c * CHUNK, CHUNK)],
            sems_w[b],
        )

    def outer(i, carry):
        for b in range(NBUF):
            chunk_step(i * NBUF + b, b)
        return carry

    lax.fori_loop(0, NCHUNK // NBUF, outer, 0, unroll=False)

    for b in range(NBUF):
        c = NCHUNK - NBUF + b
        pltpu.make_async_copy(
            rows_v.at[b],
            out_hbm.at[pl.ds(base + c * CHUNK, CHUNK)],
            sems_w[b],
        ).wait()


def kernel(word_ids, basic, modif):
    fused = jnp.concatenate([basic, modif], axis=1)
    idx = word_ids.reshape(-1)
    out = _gather(fused, idx)
    return out.reshape(BATCH, HIST, FUSED_DIM)
